# baseline (speedup 1.0000x reference)
import functools

import jax
import jax.numpy as jnp
from jax import lax
from jax.experimental import pallas as pl
from jax.experimental.pallas import tpu as pltpu
from jax.experimental.pallas import tpu_sc as plsc

NC, NS, L = 2, 16, 16
NW = NC * NS
B, S, V, D = 4, 2048, 100000, 768
N = B * S
TPW = N // NW
C = 16
NCH = TPW // C
SB = S // TPW
NV = D // L


def _body(seq_hbm, lbl_hbm, tok_hbm, seg_hbm, pe_hbm, out_hbm,
          idx_v, lbl_v, tok_v, seg_v, pe_v,
          tok_sem, seg_sem, pe_sem, out_sem):
    wid = lax.axis_index("s") * NC + lax.axis_index("c")
    sblk = lax.rem(wid, SB)

    pltpu.sync_copy(seq_hbm.at[wid], idx_v)
    pltpu.sync_copy(lbl_hbm.at[wid], lbl_v)

    in_tok = [None, None]
    in_seg = [None, None]
    in_pe = [None, None]
    out_cd = [None, None]

    def start_in(g):
        slot = g & 1
        in_tok[slot] = pltpu.async_copy(
            tok_hbm.at[idx_v.at[g]], tok_v.at[slot], tok_sem.at[slot])
        in_seg[slot] = pltpu.async_copy(
            seg_hbm.at[lbl_v.at[g]], seg_v.at[slot], seg_sem.at[slot])
        in_pe[slot] = pltpu.async_copy(
            pe_hbm.at[sblk, g], pe_v.at[slot], pe_sem.at[slot])

    def compute(slot):
        def row(i, carry):
            for j in range(NV):
                off = j * L
                t = tok_v[slot, i, pl.ds(off, L)]
                p = pe_v[slot, i, pl.ds(off, L)]
                sg = seg_v[slot, i, pl.ds(off, L)]
                tok_v[slot, i, pl.ds(off, L)] = t + p + sg
            return carry
        lax.fori_loop(0, C, row, 0)

    start_in(0)
    for g in range(NCH):
        slot = g & 1
        if g + 1 < NCH:
            if out_cd[1 - slot] is not None:
                out_cd[1 - slot].wait()
            start_in(g + 1)
        in_tok[slot].wait()
        in_seg[slot].wait()
        in_pe[slot].wait()
        compute(slot)
        out_cd[slot] = pltpu.async_copy(
            tok_v.at[slot], out_hbm.at[wid, g], out_sem.at[slot])
    out_cd[0].wait()
    out_cd[1].wait()


_sc_call = functools.partial(
    pl.kernel,
    out_type=jax.ShapeDtypeStruct((NW, NCH, C, D), jnp.float32),
    mesh=plsc.VectorSubcoreMesh(core_axis_name="c", subcore_axis_name="s"),
    scratch_types=[
        pltpu.VMEM((NCH, C), jnp.int32),
        pltpu.VMEM((NCH, C), jnp.int32),
        pltpu.VMEM((2, C, D), jnp.float32),
        pltpu.VMEM((2, C, D), jnp.float32),
        pltpu.VMEM((2, C, D), jnp.float32),
        pltpu.SemaphoreType.DMA((2,)),
        pltpu.SemaphoreType.DMA((2,)),
        pltpu.SemaphoreType.DMA((2,)),
        pltpu.SemaphoreType.DMA((2,)),
    ],
)(_body)


def kernel(sequence, segment_labels, tok_table, seg_table, pe):
    seq3 = sequence.reshape(NW, NCH, C).astype(jnp.int32)
    lbl3 = segment_labels.reshape(NW, NCH, C).astype(jnp.int32)
    pe4 = pe.reshape(S, D).reshape(SB, NCH, C, D)
    out = _sc_call(seq3, lbl3, tok_table, seg_table, pe4)
    return out.reshape(B, S, D)


# speedup vs baseline: 1.3177x; 1.3177x over previous
import functools

import jax
import jax.numpy as jnp
from jax import lax
from jax.experimental import pallas as pl
from jax.experimental.pallas import tpu as pltpu
from jax.experimental.pallas import tpu_sc as plsc

NC, NS, L = 2, 16, 16
NW = NC * NS
B, S, V, D = 4, 2048, 100000, 768
N = B * S
TPW = N // NW
C = 16
NCH = TPW // C
PR = C // B
NV = D // L
HR = 8


def _body(idx_hbm, oidx_hbm, wa_hbm, wb_hbm, segd_hbm, pe_hbm, tok_hbm,
          out_hbm, idx_v, oidx_v, wa_v, wb_v, segd_v, pe_v, tok_v, res_v,
          tok_sem, pe_sem, out_sem):
    wid = lax.axis_index("s") * NC + lax.axis_index("c")

    pltpu.sync_copy(idx_hbm.at[wid], idx_v)
    pltpu.sync_copy(oidx_hbm.at[wid], oidx_v)
    pltpu.sync_copy(wa_hbm.at[wid], wa_v)
    pltpu.sync_copy(wb_hbm.at[wid], wb_v)
    pltpu.sync_copy(segd_hbm, segd_v)

    in_tok = [None, None]
    in_pe = [None, None]
    out_cd = [None, None]

    def start_in(g):
        slot = g & 1
        in_tok[slot] = pltpu.async_copy(
            tok_hbm.at[idx_v.at[g]], tok_v.at[slot], tok_sem.at[slot])
        in_pe[slot] = pltpu.async_copy(
            pe_hbm.at[wid, g], pe_v.at[slot], pe_sem.at[slot])

    def compute(g, slot):
        for h in range(C // HR):
            r0 = g * C + h * HR
            was = tuple(wa_v[r0 + i, :] for i in range(HR))
            wbs = tuple(wb_v[r0 + i, :] for i in range(HR))

            def jbody(j, carry):
                was_, wbs_ = carry
                off = j * L
                a0 = segd_v[pl.ds(off, L)]
                a1 = segd_v[pl.ds(D + off, L)]
                a2 = segd_v[pl.ds(2 * D + off, L)]
                for i in range(HR):
                    row = h * HR + i
                    t = tok_v[slot, row, pl.ds(off, L)]
                    p = pe_v[slot, row // B, pl.ds(off, L)]
                    res_v[slot, row, pl.ds(off, L)] = (
                        t + p + a0 + was_[i] * a1 + wbs_[i] * a2)
                return was_, wbs_

            lax.fori_loop(0, NV, jbody, (was, wbs))

    start_in(0)
    for g in range(NCH):
        slot = g & 1
        if g + 1 < NCH:
            start_in(g + 1)
        in_tok[slot].wait()
        in_pe[slot].wait()
        if out_cd[slot] is not None:
            out_cd[slot].wait()
        compute(g, slot)
        out_cd[slot] = pltpu.async_copy(
            res_v.at[slot], out_hbm.at[oidx_v.at[g]], out_sem.at[slot])
    out_cd[0].wait()
    out_cd[1].wait()


_sc_call = functools.partial(
    pl.kernel,
    out_type=jax.ShapeDtypeStruct((N, D), jnp.float32),
    mesh=plsc.VectorSubcoreMesh(core_axis_name="c", subcore_axis_name="s"),
    scratch_types=[
        pltpu.VMEM((NCH, C), jnp.int32),
        pltpu.VMEM((NCH, C), jnp.int32),
        pltpu.VMEM((TPW, L), jnp.float32),
        pltpu.VMEM((TPW, L), jnp.float32),
        pltpu.VMEM((3 * D,), jnp.float32),
        pltpu.VMEM((2, PR, D), jnp.float32),
        pltpu.VMEM((2, C, D), jnp.float32),
        pltpu.VMEM((2, C, D), jnp.float32),
        pltpu.SemaphoreType.DMA((2,)),
        pltpu.SemaphoreType.DMA((2,)),
        pltpu.SemaphoreType.DMA((2,)),
    ],
)(_body)


def kernel(sequence, segment_labels, tok_table, seg_table, pe):
    seq_sm = sequence.T.reshape(NW, NCH, C).astype(jnp.int32)
    lbl_sm = segment_labels.T.reshape(NW, TPW).astype(jnp.int32)
    wa = jnp.broadcast_to(
        (lbl_sm >= 1).astype(jnp.float32)[..., None], (NW, TPW, L))
    wb = jnp.broadcast_to(
        (lbl_sm >= 2).astype(jnp.float32)[..., None], (NW, TPW, L))
    tp = jnp.arange(N, dtype=jnp.int32)
    oidx = ((tp % B) * S + tp // B).reshape(NW, NCH, C)
    segd = jnp.concatenate(
        [seg_table[0], seg_table[1] - seg_table[0],
         seg_table[2] - seg_table[1]])
    pe5 = pe.reshape(S, D).reshape(NW, NCH, PR, D)
    out = _sc_call(seq_sm, oidx, wa, wb, segd, pe5, tok_table)
    return out.reshape(B, S, D)


# speedup vs baseline: 2.5576x; 1.9410x over previous
import functools

import jax
import jax.numpy as jnp
from jax import lax
from jax.experimental import pallas as pl
from jax.experimental.pallas import tpu as pltpu
from jax.experimental.pallas import tpu_sc as plsc

NC, NS, L = 2, 16, 16
NW = NC * NS
B, S, V, D = 4, 2048, 100000, 768
N = B * S
TPW = N // NW
C = 16
NCH = TPW // C
PR = C // B
NV = D // L
HR = 8


def _body(idx_hbm, oidx_hbm, wa_hbm, wb_hbm, segd_hbm, pe_hbm, tok_hbm,
          out_hbm, idx_v, oidx_v, wa_v, wb_v, segd_v, pe_v, tok_v, res_v,
          tok_sem, pe_sem, out_sem):
    wid = lax.axis_index("s") * NC + lax.axis_index("c")

    pltpu.sync_copy(idx_hbm.at[wid], idx_v)
    pltpu.sync_copy(oidx_hbm.at[wid], oidx_v)
    pltpu.sync_copy(wa_hbm.at[wid], wa_v)
    pltpu.sync_copy(wb_hbm.at[wid], wb_v)
    pltpu.sync_copy(segd_hbm, segd_v)

    def start_in(g):
        slot = lax.rem(g, 2)
        pltpu.async_copy(
            tok_hbm.at[idx_v.at[g]], tok_v.at[slot], tok_sem.at[slot])
        pltpu.async_copy(
            pe_hbm.at[wid, g], pe_v.at[slot], pe_sem.at[slot])

    def wait_in(g, slot):
        pltpu.make_async_copy(
            tok_hbm.at[idx_v.at[g]], tok_v.at[slot], tok_sem.at[slot]).wait()
        pltpu.make_async_copy(
            pe_hbm.at[wid, g], pe_v.at[slot], pe_sem.at[slot]).wait()

    def start_out(g, slot):
        pltpu.async_copy(
            res_v.at[slot], out_hbm.at[oidx_v.at[g]], out_sem.at[slot])

    def wait_out(g, slot):
        pltpu.make_async_copy(
            res_v.at[slot], out_hbm.at[oidx_v.at[g]], out_sem.at[slot]).wait()

    def compute(g, slot):
        for h in range(C // HR):
            r0 = g * C + h * HR
            was = tuple(wa_v[r0 + i, :] for i in range(HR))
            wbs = tuple(wb_v[r0 + i, :] for i in range(HR))

            def jbody(j, carry):
                was_, wbs_ = carry
                off = j * L
                a1 = segd_v[pl.ds(off, L)]
                a2 = segd_v[pl.ds(D + off, L)]
                for i in range(HR):
                    row = h * HR + i
                    t = tok_v[slot, row, pl.ds(off, L)]
                    p = pe_v[slot, row // B, pl.ds(off, L)]
                    res_v[slot, row, pl.ds(off, L)] = (
                        t + p + was_[i] * a1 + wbs_[i] * a2)
                return was_, wbs_

            plsc.parallel_loop(0, NV, 1, unroll=2, carry=(was, wbs))(jbody)

    start_in(0)
    start_in(1)

    def gbody(g, carry):
        slot = lax.rem(g, 2)
        wait_in(g, slot)

        @pl.when(g >= 2)
        def _():
            wait_out(g - 2, slot)

        compute(g, slot)
        start_out(g, slot)

        @pl.when(g + 2 < NCH)
        def _():
            start_in(g + 2)

        return carry

    lax.fori_loop(0, NCH, gbody, 0)
    wait_out(NCH - 2, 0)
    wait_out(NCH - 1, 1)


_sc_call = functools.partial(
    pl.kernel,
    out_type=jax.ShapeDtypeStruct((N, D), jnp.float32),
    mesh=plsc.VectorSubcoreMesh(core_axis_name="c", subcore_axis_name="s"),
    scratch_types=[
        pltpu.VMEM((NCH, C), jnp.int32),
        pltpu.VMEM((NCH, C), jnp.int32),
        pltpu.VMEM((TPW, L), jnp.float32),
        pltpu.VMEM((TPW, L), jnp.float32),
        pltpu.VMEM((2 * D,), jnp.float32),
        pltpu.VMEM((2, PR, D), jnp.float32),
        pltpu.VMEM((2, C, D), jnp.float32),
        pltpu.VMEM((2, C, D), jnp.float32),
        pltpu.SemaphoreType.DMA((2,)),
        pltpu.SemaphoreType.DMA((2,)),
        pltpu.SemaphoreType.DMA((2,)),
    ],
)(_body)


def kernel(sequence, segment_labels, tok_table, seg_table, pe):
    seq_sm = sequence.T.reshape(NW, NCH, C).astype(jnp.int32)
    lbl_sm = segment_labels.T.reshape(NW, TPW).astype(jnp.int32)
    wa = jnp.broadcast_to(
        (lbl_sm >= 1).astype(jnp.float32)[..., None], (NW, TPW, L))
    wb = jnp.broadcast_to(
        (lbl_sm >= 2).astype(jnp.float32)[..., None], (NW, TPW, L))
    tp = jnp.arange(N, dtype=jnp.int32)
    oidx = ((tp % B) * S + tp // B).reshape(NW, NCH, C)
    segd = jnp.concatenate(
        [seg_table[1] - seg_table[0], seg_table[2] - seg_table[1]])
    pe5 = (pe.reshape(S, D) + seg_table[0]).reshape(NW, NCH, PR, D)
    out = _sc_call(seq_sm, oidx, wa, wb, segd, pe5, tok_table)
    return out.reshape(B, S, D)


# speedup vs baseline: 2.8331x; 1.1077x over previous
import functools

import jax
import jax.numpy as jnp
from jax import lax
from jax.experimental import pallas as pl
from jax.experimental.pallas import tpu as pltpu
from jax.experimental.pallas import tpu_sc as plsc

NC, NS, L = 2, 16, 16
NW = NC * NS
B, S, V, D = 4, 2048, 100000, 768
N = B * S
TPW = N // NW
C = 32
NCH = TPW // C
PR = C // B
NV = D // L
HR = 4


def _body(idx_hbm, oidx_hbm, w_hbm, segd_hbm, pe_hbm, tok_hbm,
          out_hbm, idx_v, oidx_v, w_v, segd_v, pe_v, tok_v, res_v,
          tok_sem, pe_sem, w_sem, out_sem, misc_sem):
    wid = lax.axis_index("s") * NC + lax.axis_index("c")

    cd_idx = pltpu.async_copy(idx_hbm.at[wid], idx_v, misc_sem)
    cd_oidx = pltpu.async_copy(oidx_hbm.at[wid], oidx_v, misc_sem)
    cd_segd = pltpu.async_copy(segd_hbm, segd_v, misc_sem)
    cd_idx.wait()

    def start_in(g):
        slot = lax.rem(g, 2)
        pltpu.async_copy(
            tok_hbm.at[idx_v.at[g]], tok_v.at[slot], tok_sem.at[slot])
        pltpu.async_copy(
            pe_hbm.at[wid, g], pe_v.at[slot], pe_sem.at[slot])
        pltpu.async_copy(
            w_hbm.at[wid, g], w_v.at[slot], w_sem.at[slot])

    def wait_in(g, slot):
        pltpu.make_async_copy(
            tok_hbm.at[idx_v.at[g]], tok_v.at[slot], tok_sem.at[slot]).wait()
        pltpu.make_async_copy(
            pe_hbm.at[wid, g], pe_v.at[slot], pe_sem.at[slot]).wait()
        pltpu.make_async_copy(
            w_hbm.at[wid, g], w_v.at[slot], w_sem.at[slot]).wait()

    def start_out(g, slot):
        pltpu.async_copy(
            res_v.at[slot], out_hbm.at[oidx_v.at[g]], out_sem.at[slot])

    def wait_out(g, slot):
        pltpu.make_async_copy(
            res_v.at[slot], out_hbm.at[oidx_v.at[g]], out_sem.at[slot]).wait()

    def compute(slot):
        for h in range(C // HR):
            was = tuple(w_v[slot, h * HR + i, 0, :] for i in range(HR))
            wbs = tuple(w_v[slot, h * HR + i, 1, :] for i in range(HR))

            def jbody(j, carry):
                was_, wbs_ = carry
                off = j * L
                a1 = segd_v[pl.ds(off, L)]
                a2 = segd_v[pl.ds(D + off, L)]
                for i in range(HR):
                    row = h * HR + i
                    t = tok_v[slot, row, pl.ds(off, L)]
                    p = pe_v[slot, row // B, pl.ds(off, L)]
                    res_v[slot, row, pl.ds(off, L)] = (
                        t + p + was_[i] * a1 + wbs_[i] * a2)
                return was_, wbs_

            plsc.parallel_loop(0, NV, 1, unroll=2, carry=(was, wbs))(jbody)

    start_in(0)
    start_in(1)
    cd_oidx.wait()
    cd_segd.wait()

    def gbody(g, carry):
        slot = lax.rem(g, 2)
        wait_in(g, slot)

        @pl.when(g >= 2)
        def _():
            wait_out(g - 2, slot)

        compute(slot)
        start_out(g, slot)

        @pl.when(g + 2 < NCH)
        def _():
            start_in(g + 2)

        return carry

    lax.fori_loop(0, NCH, gbody, 0)
    wait_out(NCH - 2, 0)
    wait_out(NCH - 1, 1)


_sc_call = functools.partial(
    pl.kernel,
    out_type=jax.ShapeDtypeStruct((N, D), jnp.float32),
    mesh=plsc.VectorSubcoreMesh(core_axis_name="c", subcore_axis_name="s"),
    scratch_types=[
        pltpu.VMEM((NCH, C), jnp.int32),
        pltpu.VMEM((NCH, C), jnp.int32),
        pltpu.VMEM((2, C, 2, L), jnp.float32),
        pltpu.VMEM((2 * D,), jnp.float32),
        pltpu.VMEM((2, PR, D), jnp.float32),
        pltpu.VMEM((2, C, D), jnp.float32),
        pltpu.VMEM((2, C, D), jnp.float32),
        pltpu.SemaphoreType.DMA((2,)),
        pltpu.SemaphoreType.DMA((2,)),
        pltpu.SemaphoreType.DMA((2,)),
        pltpu.SemaphoreType.DMA((2,)),
        pltpu.SemaphoreType.DMA,
    ],
)(_body)


def kernel(sequence, segment_labels, tok_table, seg_table, pe):
    seq_sm = sequence.T.reshape(NW, NCH, C).astype(jnp.int32)
    lbl_sm = segment_labels.T.reshape(NW, TPW).astype(jnp.int32)
    w = jnp.broadcast_to(
        jnp.stack([(lbl_sm >= 1), (lbl_sm >= 2)], axis=-1)
        .astype(jnp.float32)[..., None],
        (NW, TPW, 2, L)).reshape(NW, NCH, C, 2, L)
    tp = jnp.arange(N, dtype=jnp.int32)
    oidx = ((tp % B) * S + tp // B).reshape(NW, NCH, C)
    segd = jnp.concatenate(
        [seg_table[1] - seg_table[0], seg_table[2] - seg_table[1]])
    pe5 = (pe.reshape(S, D) + seg_table[0]).reshape(NW, NCH, PR, D)
    out = _sc_call(seq_sm, oidx, w, segd, pe5, tok_table)
    return out.reshape(B, S, D)
